# Initial kernel scaffold; baseline (speedup 1.0000x reference)
import jax
import jax.numpy as jnp
from jax.experimental import pallas as pl


def kernel(q, k, pos, pe):
    raise NotImplementedError("write your pallas kernel here")


# speedup vs baseline: 1.1089x; 1.1089x over previous
import functools

import jax
import jax.numpy as jnp
from jax import lax
from jax.experimental import pallas as pl
from jax.experimental.pallas import tpu as pltpu
from jax.experimental.pallas import tpu_sc as plsc

DIM = 1024
SEQ = 8192
BATCH = 2

NUM_WORKERS = 32
ROWS_PER_W = SEQ // NUM_WORKERS
CHUNK = 32
NCHUNK = ROWS_PER_W // CHUNK

BS = 512
NBLK = SEQ // BS


def _sc_gather(pe_hbm, pos_hbm, g_hbm, idx_v, buf0, buf1, sem0, sem1):
    wid = lax.axis_index("s") * 2 + lax.axis_index("c")
    base = wid * ROWS_PER_W
    pltpu.sync_copy(pos_hbm.at[pl.ds(base, ROWS_PER_W)], idx_v)
    bufs = (buf0, buf1)
    sems = (sem0, sem1)
    copies = []
    for c in range(NCHUNK):
        copies.append(
            pltpu.async_copy(
                pe_hbm.at[idx_v.at[pl.ds(c * CHUNK, CHUNK)]],
                bufs[c % 2],
                sems[c % 2],
            )
        )
        if c > 0:
            copies[c - 1].wait()
            pltpu.sync_copy(
                bufs[(c - 1) % 2],
                g_hbm.at[pl.ds(base + (c - 1) * CHUNK, CHUNK)],
            )
    copies[NCHUNK - 1].wait()
    pltpu.sync_copy(
        bufs[(NCHUNK - 1) % 2],
        g_hbm.at[pl.ds(base + (NCHUNK - 1) * CHUNK, CHUNK)],
    )


_gather = functools.partial(
    pl.kernel,
    out_type=jax.ShapeDtypeStruct((SEQ, DIM), jnp.float32),
    mesh=plsc.VectorSubcoreMesh(core_axis_name="c", subcore_axis_name="s"),
    scratch_types=[
        pltpu.VMEM((ROWS_PER_W,), jnp.int32),
        pltpu.VMEM((CHUNK, DIM), jnp.float32),
        pltpu.VMEM((CHUNK, DIM), jnp.float32),
        pltpu.SemaphoreType.DMA,
        pltpu.SemaphoreType.DMA,
    ],
)(_sc_gather)


def _tc_add(q_ref, k_ref, g_ref, oq_ref, ok_ref):
    g = g_ref[...]
    oq_ref[...] = q_ref[...] + g
    ok_ref[...] = k_ref[...] + g


_add = pl.pallas_call(
    _tc_add,
    grid=(BATCH, NBLK),
    in_specs=[
        pl.BlockSpec((BS, DIM), lambda b, j: (b * NBLK + j, 0)),
        pl.BlockSpec((BS, DIM), lambda b, j: (b * NBLK + j, 0)),
        pl.BlockSpec((BS, DIM), lambda b, j: (j, 0)),
    ],
    out_specs=[
        pl.BlockSpec((BS, DIM), lambda b, j: (b * NBLK + j, 0)),
        pl.BlockSpec((BS, DIM), lambda b, j: (b * NBLK + j, 0)),
    ],
    out_shape=[
        jax.ShapeDtypeStruct((BATCH * SEQ, DIM), jnp.float32),
        jax.ShapeDtypeStruct((BATCH * SEQ, DIM), jnp.float32),
    ],
)


@jax.jit
def kernel(q, k, pos, pe):
    g = _gather(pe, pos)
    q2 = q.reshape(BATCH * SEQ, DIM)
    k2 = k.reshape(BATCH * SEQ, DIM)
    oq, ok = _add(q2, k2, g)
    return oq.reshape(q.shape), ok.reshape(k.shape)


# speedup vs baseline: 1.1815x; 1.0655x over previous
import functools

import jax
import jax.numpy as jnp
from jax import lax
from jax.experimental import pallas as pl
from jax.experimental.pallas import tpu as pltpu
from jax.experimental.pallas import tpu_sc as plsc

DIM = 1024
SEQ = 8192
BATCH = 2

NSPLIT = 2
HSEQ = SEQ // NSPLIT

NUM_WORKERS = 32
ROWS_PER_W = HSEQ // NUM_WORKERS
CHUNK = 32
NCHUNK = ROWS_PER_W // CHUNK

BS = 512
HBLK = HSEQ // BS


def _sc_gather_body(off, pe_hbm, pos_hbm, g_hbm, idx_v, buf0, buf1, sem0, sem1):
    wid = lax.axis_index("s") * 2 + lax.axis_index("c")
    base = wid * ROWS_PER_W
    pltpu.sync_copy(pos_hbm.at[pl.ds(off + base, ROWS_PER_W)], idx_v)
    bufs = (buf0, buf1)
    sems = (sem0, sem1)
    copies = []
    for c in range(NCHUNK):
        copies.append(
            pltpu.async_copy(
                pe_hbm.at[idx_v.at[pl.ds(c * CHUNK, CHUNK)]],
                bufs[c % 2],
                sems[c % 2],
            )
        )
        if c > 0:
            copies[c - 1].wait()
            pltpu.sync_copy(
                bufs[(c - 1) % 2],
                g_hbm.at[pl.ds(base + (c - 1) * CHUNK, CHUNK)],
            )
    copies[NCHUNK - 1].wait()
    pltpu.sync_copy(
        bufs[(NCHUNK - 1) % 2],
        g_hbm.at[pl.ds(base + (NCHUNK - 1) * CHUNK, CHUNK)],
    )


def _make_gather(off):
    return functools.partial(
        pl.kernel,
        out_type=jax.ShapeDtypeStruct((HSEQ, DIM), jnp.float32),
        mesh=plsc.VectorSubcoreMesh(core_axis_name="c", subcore_axis_name="s"),
        scratch_types=[
            pltpu.VMEM((ROWS_PER_W,), jnp.int32),
            pltpu.VMEM((CHUNK, DIM), jnp.float32),
            pltpu.VMEM((CHUNK, DIM), jnp.float32),
            pltpu.SemaphoreType.DMA,
            pltpu.SemaphoreType.DMA,
        ],
    )(functools.partial(_sc_gather_body, off))


_gather_halves = [_make_gather(h * HSEQ) for h in range(NSPLIT)]


def _tc_add_first(q_ref, k_ref, g_ref, oq_ref, ok_ref):
    g = g_ref[...][None, :, :]
    oq_ref[...] = q_ref[...] + g
    ok_ref[...] = k_ref[...] + g


def _tc_add_next(q_ref, k_ref, g_ref, _oq_in, _ok_in, oq_ref, ok_ref):
    g = g_ref[...][None, :, :]
    oq_ref[...] = q_ref[...] + g
    ok_ref[...] = k_ref[...] + g


def _make_add(h, aliased):
    blk = lambda j: (0, h * HBLK + j, 0)
    qk_spec = pl.BlockSpec((BATCH, BS, DIM), blk)
    g_spec = pl.BlockSpec((BS, DIM), lambda j: (j, 0))
    in_specs = [qk_spec, qk_spec, g_spec]
    body = _tc_add_first
    aliases = {}
    if aliased:
        any_spec = pl.BlockSpec(memory_space=pl.ANY)
        in_specs += [any_spec, any_spec]
        body = _tc_add_next
        aliases = {3: 0, 4: 1}
    return pl.pallas_call(
        body,
        grid=(HBLK,),
        in_specs=in_specs,
        out_specs=[qk_spec, qk_spec],
        out_shape=[
            jax.ShapeDtypeStruct((BATCH, SEQ, DIM), jnp.float32),
            jax.ShapeDtypeStruct((BATCH, SEQ, DIM), jnp.float32),
        ],
        input_output_aliases=aliases,
    )


_add_halves = [_make_add(h, h > 0) for h in range(NSPLIT)]


@jax.jit
def kernel(q, k, pos, pe):
    gs = [_gather_halves[h](pe, pos) for h in range(NSPLIT)]
    oq, ok = _add_halves[0](q, k, gs[0])
    for h in range(1, NSPLIT):
        oq, ok = _add_halves[h](q, k, gs[h], oq, ok)
    return oq, ok
